# baseline (speedup 1.0000x reference)
import functools

import jax
import jax.numpy as jnp
from jax import lax
from jax.experimental import pallas as pl
from jax.experimental.pallas import tpu as pltpu
from jax.experimental.pallas import tpu_sc as plsc

B = 16384
EMB = 64
VEMB = 32
NVIS = 2048
NC = 2
NS = 16
NW = NC * NS
BPW = B // NW


def _sc_gather(u_idx, ip_idx, in_idx, user_table, item_table, uv_table):
    mesh = plsc.VectorSubcoreMesh(core_axis_name="c", subcore_axis_name="s")

    @functools.partial(
        pl.kernel,
        out_type=[
            jax.ShapeDtypeStruct((B, EMB), jnp.float32),
            jax.ShapeDtypeStruct((B, EMB), jnp.float32),
            jax.ShapeDtypeStruct((B, EMB), jnp.float32),
            jax.ShapeDtypeStruct((B, VEMB), jnp.float32),
        ],
        mesh=mesh,
        compiler_params=pltpu.CompilerParams(use_tc_tiling_on_sc=False),
        scratch_types=[
            pltpu.VMEM((BPW,), jnp.int32),
            pltpu.VMEM((BPW,), jnp.int32),
            pltpu.VMEM((BPW,), jnp.int32),
            pltpu.VMEM((BPW, EMB), jnp.float32),
            pltpu.VMEM((BPW, EMB), jnp.float32),
            pltpu.VMEM((BPW, EMB), jnp.float32),
            pltpu.VMEM((BPW, VEMB), jnp.float32),
            pltpu.SemaphoreType.DMA,
        ],
    )
    def gather_kernel(uidx_hbm, ipidx_hbm, inidx_hbm, ut_hbm, it_hbm, uvt_hbm,
                      u_out, ip_out, in_out, uv_out,
                      uidx_v, ipidx_v, inidx_v, u_v, ip_v, in_v, uv_v, sem):
        wid = lax.axis_index("s") * NC + lax.axis_index("c")
        base = wid * BPW
        pltpu.sync_copy(uidx_hbm.at[pl.ds(base, BPW)], uidx_v)
        pltpu.sync_copy(ipidx_hbm.at[pl.ds(base, BPW)], ipidx_v)
        pltpu.sync_copy(inidx_hbm.at[pl.ds(base, BPW)], inidx_v)
        c1 = pltpu.async_copy(ut_hbm.at[uidx_v], u_v, sem)
        c2 = pltpu.async_copy(it_hbm.at[ipidx_v], ip_v, sem)
        c3 = pltpu.async_copy(it_hbm.at[inidx_v], in_v, sem)
        c4 = pltpu.async_copy(uvt_hbm.at[uidx_v], uv_v, sem)
        c1.wait()
        c2.wait()
        c3.wait()
        c4.wait()
        pltpu.sync_copy(u_v, u_out.at[pl.ds(base, BPW)])
        pltpu.sync_copy(ip_v, ip_out.at[pl.ds(base, BPW)])
        pltpu.sync_copy(in_v, in_out.at[pl.ds(base, BPW)])
        pltpu.sync_copy(uv_v, uv_out.at[pl.ds(base, BPW)])

    return gather_kernel(u_idx, ip_idx, in_idx, user_table, item_table,
                         uv_table)


_BLK = 512


def _tc_score_body(vfp_ref, vfn_ref, w_ref, u_ref, ip_ref, in_ref, uv_ref,
                   pos_ref, neg_ref):
    w = w_ref[...]
    dims = (((1,), (1,)), ((), ()))
    vp = lax.dot_general(vfp_ref[...], w, dims,
                         preferred_element_type=jnp.float32)
    vn = lax.dot_general(vfn_ref[...], w, dims,
                         preferred_element_type=jnp.float32)
    u = u_ref[...]
    uv = uv_ref[...]
    pos_ref[...] = (jnp.sum(u * ip_ref[...], axis=1, keepdims=True)
                    + jnp.sum(uv * vp, axis=1, keepdims=True))
    neg_ref[...] = (jnp.sum(u * in_ref[...], axis=1, keepdims=True)
                    + jnp.sum(uv * vn, axis=1, keepdims=True))


def _tc_score(vfp, vfn, w, u, ipos, ineg, uv):
    grid = (B // _BLK,)
    pos, neg = pl.pallas_call(
        _tc_score_body,
        grid=grid,
        in_specs=[
            pl.BlockSpec((_BLK, NVIS), lambda i: (i, 0)),
            pl.BlockSpec((_BLK, NVIS), lambda i: (i, 0)),
            pl.BlockSpec((VEMB, NVIS), lambda i: (0, 0)),
            pl.BlockSpec((_BLK, EMB), lambda i: (i, 0)),
            pl.BlockSpec((_BLK, EMB), lambda i: (i, 0)),
            pl.BlockSpec((_BLK, EMB), lambda i: (i, 0)),
            pl.BlockSpec((_BLK, VEMB), lambda i: (i, 0)),
        ],
        out_specs=[
            pl.BlockSpec((_BLK, 1), lambda i: (i, 0)),
            pl.BlockSpec((_BLK, 1), lambda i: (i, 0)),
        ],
        out_shape=[
            jax.ShapeDtypeStruct((B, 1), jnp.float32),
            jax.ShapeDtypeStruct((B, 1), jnp.float32),
        ],
    )(vfp, vfn, w, u, ipos, ineg, uv)
    return pos[:, 0], neg[:, 0]


def kernel(user_indices, item_pos_indices, item_neg_indices,
           visual_features_pos, visual_features_neg,
           user_table, item_table, W_visual, user_visual_table):
    u_idx = user_indices.astype(jnp.int32)
    ip_idx = item_pos_indices.astype(jnp.int32)
    in_idx = item_neg_indices.astype(jnp.int32)
    u, ipos, ineg, uv = _sc_gather(u_idx, ip_idx, in_idx,
                                   user_table, item_table, user_visual_table)
    return _tc_score(visual_features_pos, visual_features_neg, W_visual,
                     u, ipos, ineg, uv)


# speedup vs baseline: 1.0023x; 1.0023x over previous
import functools

import jax
import jax.numpy as jnp
from jax import lax
from jax.experimental import pallas as pl
from jax.experimental.pallas import tpu as pltpu
from jax.experimental.pallas import tpu_sc as plsc

B = 16384
EMB = 64
VEMB = 32
NVIS = 2048
NC = 2
NS = 16
NW = NC * NS
BPW = B // NW
NGRP = BPW // 16


def _sc_gather_dots(u_idx, ip_idx, in_idx, user_table, item_table, uv_table):
    mesh = plsc.VectorSubcoreMesh(core_axis_name="c", subcore_axis_name="s")

    @functools.partial(
        pl.kernel,
        out_type=[
            jax.ShapeDtypeStruct((B,), jnp.float32),
            jax.ShapeDtypeStruct((B,), jnp.float32),
            jax.ShapeDtypeStruct((B, VEMB), jnp.float32),
        ],
        mesh=mesh,
        compiler_params=pltpu.CompilerParams(use_tc_tiling_on_sc=False,
                                             needs_layout_passes=False),
        scratch_types=[
            pltpu.VMEM((BPW,), jnp.int32),
            pltpu.VMEM((BPW,), jnp.int32),
            pltpu.VMEM((BPW,), jnp.int32),
            pltpu.VMEM((BPW, EMB), jnp.float32),
            pltpu.VMEM((BPW, EMB), jnp.float32),
            pltpu.VMEM((BPW, EMB), jnp.float32),
            pltpu.VMEM((BPW, VEMB), jnp.float32),
            pltpu.VMEM((BPW,), jnp.float32),
            pltpu.VMEM((BPW,), jnp.float32),
            pltpu.SemaphoreType.DMA,
        ],
    )
    def body(uidx_hbm, ipidx_hbm, inidx_hbm, ut_hbm, it_hbm, uvt_hbm,
             gp_out, gn_out, uv_out,
             uidx_v, ipidx_v, inidx_v, u_v, ip_v, in_v, uv_v, gp_v, gn_v,
             sem):
        wid = lax.axis_index("s") * NC + lax.axis_index("c")
        base = wid * BPW
        pltpu.sync_copy(uidx_hbm.at[pl.ds(base, BPW)], uidx_v)
        pltpu.sync_copy(ipidx_hbm.at[pl.ds(base, BPW)], ipidx_v)
        pltpu.sync_copy(inidx_hbm.at[pl.ds(base, BPW)], inidx_v)
        c1 = pltpu.async_copy(ut_hbm.at[uidx_v], u_v, sem)
        c2 = pltpu.async_copy(it_hbm.at[ipidx_v], ip_v, sem)
        c3 = pltpu.async_copy(it_hbm.at[inidx_v], in_v, sem)
        c4 = pltpu.async_copy(uvt_hbm.at[uidx_v], uv_v, sem)
        c1.wait()
        c2.wait()
        c3.wait()
        c4.wait()

        for g in range(NGRP):
            rows = g * 16 + lax.iota(jnp.int32, 16)

            def dot_body(d, carry):
                accp, accn = carry
                cols = jnp.full((16,), d, jnp.int32)
                u16 = plsc.load_gather(u_v, [rows, cols])
                accp = accp + u16 * plsc.load_gather(ip_v, [rows, cols])
                accn = accn + u16 * plsc.load_gather(in_v, [rows, cols])
                return accp, accn

            zeros = jnp.zeros((16,), jnp.float32)
            accp, accn = lax.fori_loop(0, EMB, dot_body, (zeros, zeros))
            gp_v[pl.ds(g * 16, 16)] = accp
            gn_v[pl.ds(g * 16, 16)] = accn

        pltpu.sync_copy(gp_v, gp_out.at[pl.ds(base, BPW)])
        pltpu.sync_copy(gn_v, gn_out.at[pl.ds(base, BPW)])
        pltpu.sync_copy(uv_v, uv_out.at[pl.ds(base, BPW)])

    return body(u_idx, ip_idx, in_idx, user_table, item_table, uv_table)


_BLK = 1024


def _tc_matmul_body(vfp_ref, vfn_ref, w_ref, vp_ref, vn_ref):
    w = w_ref[...]
    dims = (((1,), (1,)), ((), ()))
    vp_ref[...] = lax.dot_general(vfp_ref[...], w, dims,
                                  preferred_element_type=jnp.float32)
    vn_ref[...] = lax.dot_general(vfn_ref[...], w, dims,
                                  preferred_element_type=jnp.float32)


def _tc_matmul(vfp, vfn, w):
    return pl.pallas_call(
        _tc_matmul_body,
        grid=(B // _BLK,),
        in_specs=[
            pl.BlockSpec((_BLK, NVIS), lambda i: (i, 0)),
            pl.BlockSpec((_BLK, NVIS), lambda i: (i, 0)),
            pl.BlockSpec((VEMB, NVIS), lambda i: (0, 0)),
        ],
        out_specs=[
            pl.BlockSpec((_BLK, VEMB), lambda i: (i, 0)),
            pl.BlockSpec((_BLK, VEMB), lambda i: (i, 0)),
        ],
        out_shape=[
            jax.ShapeDtypeStruct((B, VEMB), jnp.float32),
            jax.ShapeDtypeStruct((B, VEMB), jnp.float32),
        ],
    )(vfp, vfn, w)


def _tc_combine_body(vp_ref, vn_ref, uv_ref, gp_ref, gn_ref,
                     pos_ref, neg_ref):
    uv = uv_ref[...]
    pos_ref[...] = gp_ref[...] + jnp.sum(uv * vp_ref[...], axis=1,
                                         keepdims=True)
    neg_ref[...] = gn_ref[...] + jnp.sum(uv * vn_ref[...], axis=1,
                                         keepdims=True)


def _tc_combine(vp, vn, uv, gp, gn):
    pos, neg = pl.pallas_call(
        _tc_combine_body,
        grid=(B // _BLK,),
        in_specs=[
            pl.BlockSpec((_BLK, VEMB), lambda i: (i, 0)),
            pl.BlockSpec((_BLK, VEMB), lambda i: (i, 0)),
            pl.BlockSpec((_BLK, VEMB), lambda i: (i, 0)),
            pl.BlockSpec((_BLK, 1), lambda i: (i, 0)),
            pl.BlockSpec((_BLK, 1), lambda i: (i, 0)),
        ],
        out_specs=[
            pl.BlockSpec((_BLK, 1), lambda i: (i, 0)),
            pl.BlockSpec((_BLK, 1), lambda i: (i, 0)),
        ],
        out_shape=[
            jax.ShapeDtypeStruct((B, 1), jnp.float32),
            jax.ShapeDtypeStruct((B, 1), jnp.float32),
        ],
    )(vp, vn, uv, gp.reshape(B, 1), gn.reshape(B, 1))
    return pos[:, 0], neg[:, 0]


def kernel(user_indices, item_pos_indices, item_neg_indices,
           visual_features_pos, visual_features_neg,
           user_table, item_table, W_visual, user_visual_table):
    u_idx = user_indices.astype(jnp.int32)
    ip_idx = item_pos_indices.astype(jnp.int32)
    in_idx = item_neg_indices.astype(jnp.int32)
    gp, gn, uv = _sc_gather_dots(u_idx, ip_idx, in_idx,
                                 user_table, item_table, user_visual_table)
    vp, vn = _tc_matmul(visual_features_pos, visual_features_neg, W_visual)
    return _tc_combine(vp, vn, uv, gp, gn)
